# baseline (speedup 1.0000x reference)
import numpy as np
import jax
import jax.numpy as jnp
from jax.experimental import pallas as pl
from jax.experimental.pallas import tpu as pltpu

_ROTS = ((13, 15, 26, 6), (17, 29, 16, 24))
_TINY = np.float32(np.finfo(np.float32).tiny)
_SPAN = np.float32(np.float32(1.0) - _TINY)


def _threefry2x32_np(k1, k2, x0, x1):
    k1 = np.uint32(k1)
    k2 = np.uint32(k2)
    x0 = x0.astype(np.uint32).copy()
    x1 = x1.astype(np.uint32).copy()
    ks = (k1, k2, np.uint32(k1 ^ k2 ^ np.uint32(0x1BD11BDA)))
    x0 = x0 + ks[0]
    x1 = x1 + ks[1]
    for i in range(5):
        for r in _ROTS[i % 2]:
            x0 = x0 + x1
            x1 = ((x1 << np.uint32(r)) | (x1 >> np.uint32(32 - r))).astype(np.uint32)
            x1 = x1 ^ x0
        x0 = x0 + ks[(i + 1) % 3]
        x1 = x1 + ks[(i + 2) % 3] + np.uint32(i + 1)
    return x0, x1


def _fold_in_keys(seed, n):
    i = np.arange(n, dtype=np.uint32)
    w0, w1 = _threefry2x32_np(np.uint32(0), np.uint32(seed),
                              np.zeros(n, np.uint32), i)
    return [(int(w0[d]), int(w1[d])) for d in range(n)]


_KEYS = _fold_in_keys(42, 4)


def _gumbel_bits(lo, k1, k2):
    ks = (np.uint32(k1), np.uint32(k2),
          np.uint32(np.uint32(k1) ^ np.uint32(k2) ^ np.uint32(0x1BD11BDA)))
    x0 = jnp.full(lo.shape, ks[0], jnp.uint32)
    x1 = lo + ks[1]
    for i in range(5):
        for r in _ROTS[i % 2]:
            x0 = x0 + x1
            x1 = (x1 << np.uint32(r)) | (x1 >> np.uint32(32 - r))
            x1 = x1 ^ x0
        x0 = x0 + ks[(i + 1) % 3]
        x1 = x1 + ks[(i + 2) % 3] + np.uint32(i + 1)
    return x0 ^ x1


def _body(x_ref, o_ref, accv, acci, *, vb, nb, rows, vocab):
    i = pl.program_id(0)
    j = pl.program_id(1)

    @pl.when(j == 0)
    def _init():
        accv[...] = jnp.full_like(accv[...], -jnp.inf)
        acci[...] = jnp.zeros_like(acci[...])

    x = x_ref[...]
    col = jax.lax.broadcasted_iota(jnp.int32, (rows, vb), 1) + j * vb
    row = jax.lax.broadcasted_iota(jnp.int32, (rows, vb), 0) + i * rows
    flat = (row * vocab + col).astype(jnp.uint32)
    x = jnp.where(col < vocab, x, -jnp.inf)

    sub = 256
    av = [accv[d] for d in range(4)]
    ai = [acci[d] for d in range(4)]
    for s0 in range(0, vb, sub):
        xs = x[:, s0:s0 + sub]
        fs = flat[:, s0:s0 + sub]
        for d in range(4):
            k1, k2 = _KEYS[d]
            bits = _gumbel_bits(fs, k1, k2)
            mant = (bits >> np.uint32(9)) | np.uint32(0x3F800000)
            f = jax.lax.bitcast_convert_type(mant, jnp.float32) - jnp.float32(1.0)
            u = jnp.maximum(_TINY, f * _SPAN + _TINY)
            g = -jnp.log(-jnp.log(u))
            s = xs + g
            for c in range(sub // 128):
                sc = s[:, c * 128:(c + 1) * 128]
                base = j * vb + s0 + c * 128
                better = sc > av[d]
                av[d] = jnp.where(better, sc, av[d])
                ai[d] = jnp.where(better, base, ai[d])
    for d in range(4):
        accv[d] = av[d]
        acci[d] = ai[d]

    @pl.when(j == nb - 1)
    def _flush():
        lane = jax.lax.broadcasted_iota(jnp.int32, (rows, 128), 1)
        for d in range(4):
            av = accv[d]
            ai = acci[d]
            m = jnp.max(av, axis=-1, keepdims=True)
            cand = jnp.where(av == m, ai + lane, jnp.int32(np.iinfo(np.int32).max))
            o_ref[0, d, :] = jnp.min(cand, axis=-1)


def kernel(logits):
    b, vocab = logits.shape
    row_split = 2
    rows = b // row_split
    vb = 4096
    nb = pl.cdiv(vocab, vb)

    out = pl.pallas_call(
        lambda x_ref, o_ref, accv, acci: _body(
            x_ref, o_ref, accv, acci, vb=vb, nb=nb, rows=rows, vocab=vocab),
        grid=(row_split, nb),
        in_specs=[pl.BlockSpec((rows, vb), lambda i, j: (i, j))],
        out_specs=pl.BlockSpec((1, 4, rows), lambda i, j: (i, 0, 0)),
        out_shape=jax.ShapeDtypeStruct((row_split, 4, rows), jnp.int32),
        scratch_shapes=[
            pltpu.VMEM((4, rows, 128), jnp.float32),
            pltpu.VMEM((4, rows, 128), jnp.int32),
        ],
        compiler_params=pltpu.CompilerParams(
            dimension_semantics=("parallel", "arbitrary"),
        ),
    )(logits)
    return out.transpose(1, 0, 2).reshape(4, b)


# speedup vs baseline: 1.0090x; 1.0090x over previous
import numpy as np
import jax
import jax.numpy as jnp
from jax.experimental import pallas as pl
from jax.experimental.pallas import tpu as pltpu

_ROTS = ((13, 15, 26, 6), (17, 29, 16, 24))
_TINY = np.float32(np.finfo(np.float32).tiny)
_SPAN = np.float32(np.float32(1.0) - _TINY)


def _threefry2x32_np(k1, k2, x0, x1):
    k1 = np.uint32(k1)
    k2 = np.uint32(k2)
    x0 = x0.astype(np.uint32).copy()
    x1 = x1.astype(np.uint32).copy()
    ks = (k1, k2, np.uint32(k1 ^ k2 ^ np.uint32(0x1BD11BDA)))
    x0 = x0 + ks[0]
    x1 = x1 + ks[1]
    for i in range(5):
        for r in _ROTS[i % 2]:
            x0 = x0 + x1
            x1 = ((x1 << np.uint32(r)) | (x1 >> np.uint32(32 - r))).astype(np.uint32)
            x1 = x1 ^ x0
        x0 = x0 + ks[(i + 1) % 3]
        x1 = x1 + ks[(i + 2) % 3] + np.uint32(i + 1)
    return x0, x1


def _fold_in_keys(seed, n):
    i = np.arange(n, dtype=np.uint32)
    w0, w1 = _threefry2x32_np(np.uint32(0), np.uint32(seed),
                              np.zeros(n, np.uint32), i)
    return [(int(w0[d]), int(w1[d])) for d in range(n)]


_KEYS = _fold_in_keys(42, 4)


def _gumbel_bits(lo, k1, k2):
    ks = (np.uint32(k1), np.uint32(k2),
          np.uint32(np.uint32(k1) ^ np.uint32(k2) ^ np.uint32(0x1BD11BDA)))
    x0 = jnp.full(lo.shape, ks[0], jnp.uint32)
    x1 = lo + ks[1]
    for i in range(5):
        for r in _ROTS[i % 2]:
            x0 = x0 + x1
            x1 = (x1 << np.uint32(r)) | (x1 >> np.uint32(32 - r))
            x1 = x1 ^ x0
        x0 = x0 + ks[(i + 1) % 3]
        x1 = x1 + ks[(i + 2) % 3] + np.uint32(i + 1)
    return x0 ^ x1


def _body(x_ref, o_ref, accv, acci, *, vb, nb, rows, vocab):
    i = pl.program_id(0)
    j = pl.program_id(1)

    @pl.when(j == 0)
    def _init():
        accv[...] = jnp.full_like(accv[...], -jnp.inf)
        acci[...] = jnp.zeros_like(acci[...])

    x = x_ref[...]
    col = jax.lax.broadcasted_iota(jnp.int32, (rows, vb), 1) + j * vb
    row = jax.lax.broadcasted_iota(jnp.int32, (rows, vb), 0) + i * rows
    flat = (row * vocab + col).astype(jnp.uint32)
    x = jnp.where(col < vocab, x, -jnp.inf)

    sub = 512
    av = [accv[d] for d in range(4)]
    ai = [acci[d] for d in range(4)]
    for s0 in range(0, vb, sub):
        xs = x[:, s0:s0 + sub]
        fs = flat[:, s0:s0 + sub]
        for d in range(4):
            k1, k2 = _KEYS[d]
            bits = _gumbel_bits(fs, k1, k2)
            mant = (bits >> np.uint32(9)) | np.uint32(0x3F800000)
            f = jax.lax.bitcast_convert_type(mant, jnp.float32) - jnp.float32(1.0)
            u = jnp.maximum(_TINY, f)
            g = -jnp.log(-jnp.log(u))
            s = xs + g
            for c in range(sub // 128):
                sc = s[:, c * 128:(c + 1) * 128]
                base = j * vb + s0 + c * 128
                better = sc > av[d]
                av[d] = jnp.where(better, sc, av[d])
                ai[d] = jnp.where(better, base, ai[d])
    for d in range(4):
        accv[d] = av[d]
        acci[d] = ai[d]

    @pl.when(j == nb - 1)
    def _flush():
        lane = jax.lax.broadcasted_iota(jnp.int32, (rows, 128), 1)
        for d in range(4):
            av = accv[d]
            ai = acci[d]
            m = jnp.max(av, axis=-1, keepdims=True)
            cand = jnp.where(av == m, ai + lane, jnp.int32(np.iinfo(np.int32).max))
            o_ref[0, d, :] = jnp.min(cand, axis=-1)


def kernel(logits):
    b, vocab = logits.shape
    row_split = 2
    rows = b // row_split
    vb = 4096
    nb = pl.cdiv(vocab, vb)

    out = pl.pallas_call(
        lambda x_ref, o_ref, accv, acci: _body(
            x_ref, o_ref, accv, acci, vb=vb, nb=nb, rows=rows, vocab=vocab),
        grid=(row_split, nb),
        in_specs=[pl.BlockSpec((rows, vb), lambda i, j: (i, j))],
        out_specs=pl.BlockSpec((1, 4, rows), lambda i, j: (i, 0, 0)),
        out_shape=jax.ShapeDtypeStruct((row_split, 4, rows), jnp.int32),
        scratch_shapes=[
            pltpu.VMEM((4, rows, 128), jnp.float32),
            pltpu.VMEM((4, rows, 128), jnp.int32),
        ],
        compiler_params=pltpu.CompilerParams(
            dimension_semantics=("parallel", "arbitrary"),
        ),
    )(logits)
    return out.transpose(1, 0, 2).reshape(4, b)
